# baseline (speedup 1.0000x reference)
import functools

import jax
import jax.numpy as jnp
from jax import lax
from jax.experimental import pallas as pl
from jax.experimental.pallas import tpu as pltpu
from jax.experimental.pallas import tpu_sc as plsc

N = 1600000
B = 10000
S = 100
NC = 2
NS = 16
NW = NC * NS
PT = N // NW
CHUNK = 10000
NCHUNK = PT // CHUNK
VPC = CHUNK // 16
STRIDE = VPC
UNROLL = 25
BPAD = 10240
COLS = BPAD // NS
CV = COLS // 16
SPAD = 112

_mesh = plsc.VectorSubcoreMesh(core_axis_name="c", subcore_axis_name="s")


@functools.partial(
    pl.kernel,
    mesh=_mesh,
    compiler_params=pltpu.CompilerParams(needs_layout_passes=False),
    out_type=jax.ShapeDtypeStruct((NW, BPAD), jnp.float32),
    scratch_types=[
        pltpu.VMEM((CHUNK,), jnp.float32),
        pltpu.VMEM((CHUNK,), jnp.float32),
        pltpu.VMEM((CHUNK,), jnp.int32),
        pltpu.VMEM((CHUNK,), jnp.int32),
        pltpu.VMEM((CHUNK,), jnp.int32),
        pltpu.VMEM((CHUNK,), jnp.int32),
        pltpu.VMEM((SPAD,), jnp.float32),
        pltpu.VMEM((SPAD,), jnp.float32),
        pltpu.VMEM((SPAD,), jnp.int32),
        pltpu.VMEM((BPAD,), jnp.float32),
        pltpu.SemaphoreType.DMA,
        pltpu.SemaphoreType.DMA,
    ],
)
def _sc_segsum(e_hbm, sp_hbm, id_hbm, scale_hbm, shift_hbm, out_hbm,
               e_v0, e_v1, sp_v0, sp_v1, id_v0, id_v1,
               scale_v, shift_v, packed_v, acc_v, sem0, sem1):
    c = lax.axis_index("c")
    s = lax.axis_index("s")
    wid = s * NC + c
    base = wid * PT
    sems = (sem0, sem1)
    bufs = ((e_v0, sp_v0, id_v0), (e_v1, sp_v1, id_v1))

    def start_chunk(b, ch):
        off = base + ch * CHUNK
        e_b, sp_b, id_b = bufs[b]
        return [
            pltpu.async_copy(e_hbm.at[pl.ds(off, CHUNK)], e_b, sems[b]),
            pltpu.async_copy(sp_hbm.at[pl.ds(off, CHUNK)], sp_b, sems[b]),
            pltpu.async_copy(id_hbm.at[pl.ds(off, CHUNK)], id_b, sems[b]),
        ]

    stride_iota = lax.iota(jnp.int32, 16) * STRIDE

    def compute_chunk(b):
        e_b, sp_b, id_b = bufs[b]

        @plsc.parallel_loop(0, VPC, unroll=UNROLL)
        def _vec(i):
            idx = stride_iota + i
            e16 = plsc.load_gather(e_b, [idx])
            s16 = plsc.load_gather(sp_b, [idx])
            i16 = plsc.load_gather(id_b, [idx])
            g16 = plsc.load_gather(packed_v, [s16])
            sc16 = plsc.bitcast(lax.shift_left(g16, 16), jnp.float32)
            sh16 = plsc.bitcast(g16 & jnp.int32(-65536), jnp.float32)
            plsc.addupdate_scatter(acc_v, [i16], sc16 * e16 + sh16)

    pending = {0: start_chunk(0, 0), 1: None}

    zeros16 = jnp.zeros((16,), jnp.float32)

    @plsc.parallel_loop(0, BPAD // 16, unroll=8)
    def _zero(i):
        acc_v[pl.ds(i * 16, 16)] = zeros16

    pltpu.sync_copy(scale_hbm, scale_v.at[pl.ds(0, S)])
    pltpu.sync_copy(shift_hbm, shift_v.at[pl.ds(0, S)])

    @plsc.parallel_loop(0, SPAD // 16, unroll=7)
    def _pack(i):
        o = i * 16
        scb = plsc.bitcast(scale_v[pl.ds(o, 16)], jnp.int32)
        shb = plsc.bitcast(shift_v[pl.ds(o, 16)], jnp.int32)
        scr = scb + 0x7FFF + (lax.shift_right_logical(scb, 16) & 1)
        shr = shb + 0x7FFF + (lax.shift_right_logical(shb, 16) & 1)
        packed_v[pl.ds(o, 16)] = (
            lax.shift_right_logical(scr, 16)
            | (shr & jnp.int32(-65536))
        )

    for ch in range(NCHUNK):
        b = ch % 2
        for cp in pending[b]:
            cp.wait()
        if ch + 1 < NCHUNK:
            pending[1 - b] = start_chunk(1 - b, ch + 1)
        compute_chunk(b)

    pltpu.sync_copy(acc_v, out_hbm.at[wid])


def _tc_add_body(p_ref, o_ref):
    acc = p_ref[0]
    for r in range(1, NW):
        acc = acc + p_ref[r]
    o_ref[...] = acc[:B]


_tc_add = pl.pallas_call(
    _tc_add_body,
    out_shape=jax.ShapeDtypeStruct((B,), jnp.float32),
)


def kernel(local_energies, species, segment_ids, num_segments, scale, shift):
    del num_segments
    partials = _sc_segsum(
        local_energies,
        species.astype(jnp.int32),
        segment_ids.astype(jnp.int32),
        scale,
        shift,
    )
    return _tc_add(partials)


# speedup vs baseline: 1.0589x; 1.0589x over previous
import functools

import jax
import jax.numpy as jnp
from jax import lax
from jax.experimental import pallas as pl
from jax.experimental.pallas import tpu as pltpu
from jax.experimental.pallas import tpu_sc as plsc

N = 1600000
B = 10000
S = 100
NC = 2
NS = 16
NW = NC * NS
PT = N // NW
CHUNK = 10000
NCHUNK = PT // CHUNK
VPC = CHUNK // 16
STRIDE = VPC
UNROLL = 5
BPAD = 10240
COLS = BPAD // NS
CV = COLS // 16
SPAD = 112

_mesh = plsc.VectorSubcoreMesh(core_axis_name="c", subcore_axis_name="s")


@functools.partial(
    pl.kernel,
    mesh=_mesh,
    compiler_params=pltpu.CompilerParams(needs_layout_passes=False),
    out_type=jax.ShapeDtypeStruct((NW, BPAD), jnp.float32),
    scratch_types=[
        pltpu.VMEM((CHUNK,), jnp.float32),
        pltpu.VMEM((CHUNK,), jnp.float32),
        pltpu.VMEM((CHUNK,), jnp.int32),
        pltpu.VMEM((CHUNK,), jnp.int32),
        pltpu.VMEM((CHUNK,), jnp.int32),
        pltpu.VMEM((CHUNK,), jnp.int32),
        pltpu.VMEM((SPAD,), jnp.float32),
        pltpu.VMEM((SPAD,), jnp.float32),
        pltpu.VMEM((SPAD,), jnp.int32),
        pltpu.VMEM((BPAD,), jnp.float32),
        pltpu.SemaphoreType.DMA,
        pltpu.SemaphoreType.DMA,
    ],
)
def _sc_segsum(e_hbm, sp_hbm, id_hbm, scale_hbm, shift_hbm, out_hbm,
               e_v0, e_v1, sp_v0, sp_v1, id_v0, id_v1,
               scale_v, shift_v, packed_v, acc_v, sem0, sem1):
    c = lax.axis_index("c")
    s = lax.axis_index("s")
    wid = s * NC + c
    base = wid * PT
    sems = (sem0, sem1)
    bufs = ((e_v0, sp_v0, id_v0), (e_v1, sp_v1, id_v1))

    def start_chunk(b, ch):
        off = base + ch * CHUNK
        e_b, sp_b, id_b = bufs[b]
        return [
            pltpu.async_copy(e_hbm.at[pl.ds(off, CHUNK)], e_b, sems[b]),
            pltpu.async_copy(sp_hbm.at[pl.ds(off, CHUNK)], sp_b, sems[b]),
            pltpu.async_copy(id_hbm.at[pl.ds(off, CHUNK)], id_b, sems[b]),
        ]

    stride_iota = lax.iota(jnp.int32, 16) * STRIDE

    def compute_chunk(b):
        e_b, sp_b, id_b = bufs[b]

        @plsc.parallel_loop(0, VPC, unroll=UNROLL)
        def _vec(i):
            idx = stride_iota + i
            e16 = plsc.load_gather(e_b, [idx])
            s16 = plsc.load_gather(sp_b, [idx])
            i16 = plsc.load_gather(id_b, [idx])
            g16 = plsc.load_gather(packed_v, [s16])
            sc16 = plsc.bitcast(lax.shift_left(g16, 16), jnp.float32)
            sh16 = plsc.bitcast(g16 & jnp.int32(-65536), jnp.float32)
            plsc.addupdate_scatter(acc_v, [i16], sc16 * e16 + sh16)

    pending = {0: start_chunk(0, 0), 1: None}

    zeros16 = jnp.zeros((16,), jnp.float32)

    @plsc.parallel_loop(0, BPAD // 16, unroll=8)
    def _zero(i):
        acc_v[pl.ds(i * 16, 16)] = zeros16

    pltpu.sync_copy(scale_hbm, scale_v.at[pl.ds(0, S)])
    pltpu.sync_copy(shift_hbm, shift_v.at[pl.ds(0, S)])

    @plsc.parallel_loop(0, SPAD // 16, unroll=7)
    def _pack(i):
        o = i * 16
        scb = plsc.bitcast(scale_v[pl.ds(o, 16)], jnp.int32)
        shb = plsc.bitcast(shift_v[pl.ds(o, 16)], jnp.int32)
        scr = scb + 0x7FFF + (lax.shift_right_logical(scb, 16) & 1)
        shr = shb + 0x7FFF + (lax.shift_right_logical(shb, 16) & 1)
        packed_v[pl.ds(o, 16)] = (
            lax.shift_right_logical(scr, 16)
            | (shr & jnp.int32(-65536))
        )

    for ch in range(NCHUNK):
        b = ch % 2
        for cp in pending[b]:
            cp.wait()
        if ch + 1 < NCHUNK:
            pending[1 - b] = start_chunk(1 - b, ch + 1)
        compute_chunk(b)

    pltpu.sync_copy(acc_v, out_hbm.at[wid])


def _tc_add_body(p_ref, o_ref):
    acc = p_ref[0]
    for r in range(1, NW):
        acc = acc + p_ref[r]
    o_ref[...] = acc[:B]


_tc_add = pl.pallas_call(
    _tc_add_body,
    out_shape=jax.ShapeDtypeStruct((B,), jnp.float32),
)


def kernel(local_energies, species, segment_ids, num_segments, scale, shift):
    del num_segments
    partials = _sc_segsum(
        local_energies,
        species.astype(jnp.int32),
        segment_ids.astype(jnp.int32),
        scale,
        shift,
    )
    return _tc_add(partials)


# speedup vs baseline: 1.0657x; 1.0064x over previous
import functools

import jax
import jax.numpy as jnp
from jax import lax
from jax.experimental import pallas as pl
from jax.experimental.pallas import tpu as pltpu
from jax.experimental.pallas import tpu_sc as plsc

N = 1600000
B = 10000
S = 100
NC = 2
NS = 16
NW = NC * NS
PT = N // NW
CHUNK = 10000
NCHUNK = PT // CHUNK
VPC = CHUNK // 16
STRIDE = VPC
UNROLL = 5
BPAD = 10240
COLS = BPAD // NS
CV = COLS // 16
SPAD = 112

_mesh = plsc.VectorSubcoreMesh(core_axis_name="c", subcore_axis_name="s")


@functools.partial(
    pl.kernel,
    mesh=_mesh,
    compiler_params=pltpu.CompilerParams(needs_layout_passes=False),
    out_type=jax.ShapeDtypeStruct((NW, BPAD), jnp.float32),
    scratch_types=[
        pltpu.VMEM((CHUNK,), jnp.float32),
        pltpu.VMEM((CHUNK,), jnp.float32),
        pltpu.VMEM((CHUNK,), jnp.int32),
        pltpu.VMEM((CHUNK,), jnp.int32),
        pltpu.VMEM((CHUNK,), jnp.int32),
        pltpu.VMEM((CHUNK,), jnp.int32),
        pltpu.VMEM((SPAD,), jnp.float32),
        pltpu.VMEM((SPAD,), jnp.float32),
        pltpu.VMEM((SPAD,), jnp.int32),
        pltpu.VMEM((BPAD,), jnp.float32),
        pltpu.SemaphoreType.DMA,
        pltpu.SemaphoreType.DMA,
    ],
)
def _sc_segsum(e_hbm, sp_hbm, id_hbm, scale_hbm, shift_hbm, out_hbm,
               e_v0, e_v1, sp_v0, sp_v1, id_v0, id_v1,
               scale_v, shift_v, packed_v, acc_v, sem0, sem1):
    c = lax.axis_index("c")
    s = lax.axis_index("s")
    wid = s * NC + c
    base = wid * PT
    sems = (sem0, sem1)
    bufs = ((e_v0, sp_v0, id_v0), (e_v1, sp_v1, id_v1))

    def start_chunk(b, ch):
        off = base + ch * CHUNK
        e_b, sp_b, id_b = bufs[b]
        return [
            pltpu.async_copy(e_hbm.at[pl.ds(off, CHUNK)], e_b, sems[b]),
            pltpu.async_copy(sp_hbm.at[pl.ds(off, CHUNK)], sp_b, sems[b]),
            pltpu.async_copy(id_hbm.at[pl.ds(off, CHUNK)], id_b, sems[b]),
        ]

    stride_iota = lax.iota(jnp.int32, 16) * STRIDE

    def compute_chunk(b):
        e_b, sp_b, id_b = bufs[b]

        @plsc.parallel_loop(0, VPC, unroll=UNROLL)
        def _vec(i):
            idx = stride_iota + i
            e16 = plsc.load_gather(e_b, [idx])
            s16 = plsc.load_gather(sp_b, [idx])
            i16 = plsc.load_gather(id_b, [idx])
            g16 = plsc.load_gather(packed_v, [s16])
            sc16 = plsc.bitcast(lax.shift_left(g16, 16), jnp.float32)
            sh16 = plsc.bitcast(g16 & jnp.int32(-65536), jnp.float32)
            plsc.addupdate_scatter(acc_v, [i16], sc16 * e16 + sh16)

    pending = {0: start_chunk(0, 0), 1: None}

    zeros16 = jnp.zeros((16,), jnp.float32)

    @plsc.parallel_loop(0, BPAD // 16, unroll=8)
    def _zero(i):
        acc_v[pl.ds(i * 16, 16)] = zeros16

    pltpu.sync_copy(scale_hbm, scale_v.at[pl.ds(0, S)])
    pltpu.sync_copy(shift_hbm, shift_v.at[pl.ds(0, S)])

    @plsc.parallel_loop(0, SPAD // 16, unroll=7)
    def _pack(i):
        o = i * 16
        scb = plsc.bitcast(scale_v[pl.ds(o, 16)], jnp.int32)
        shb = plsc.bitcast(shift_v[pl.ds(o, 16)], jnp.int32)
        scr = scb + 0x7FFF + (lax.shift_right_logical(scb, 16) & 1)
        shr = shb + 0x7FFF + (lax.shift_right_logical(shb, 16) & 1)
        packed_v[pl.ds(o, 16)] = (
            lax.shift_right_logical(scr, 16)
            | (shr & jnp.int32(-65536))
        )

    for ch in range(NCHUNK):
        b = ch % 2
        for cp in pending[b]:
            cp.wait()
        if ch + 1 < NCHUNK:
            pending[1 - b] = start_chunk(1 - b, ch + 1)
        compute_chunk(b)

    pltpu.sync_copy(acc_v, out_hbm.at[wid])


def _tc_add_body(p_ref, o_ref):
    o_ref[...] = jnp.sum(p_ref[...], axis=0)[:B]


_tc_add = pl.pallas_call(
    _tc_add_body,
    out_shape=jax.ShapeDtypeStruct((B,), jnp.float32),
)


def kernel(local_energies, species, segment_ids, num_segments, scale, shift):
    del num_segments
    partials = _sc_segsum(
        local_energies,
        species.astype(jnp.int32),
        segment_ids.astype(jnp.int32),
        scale,
        shift,
    )
    return _tc_add(partials)
